# baseline (speedup 1.0000x reference)
import functools

import jax
import jax.numpy as jnp
from jax import lax
from jax.experimental import pallas as pl
from jax.experimental.pallas import tpu as pltpu
from jax.experimental.pallas import tpu_sc as plsc

NC = 2
NS = 16
NW = NC * NS
LANE = 16


def _detile_table_tc(tt):
    D, V = tt.shape
    VB = 2048
    grid = (V + VB - 1) // VB

    PACK = 128 // D

    def body(i_ref, o_ref):
        idx = lax.iota(jnp.int32, 128 // PACK) * PACK
        for g in range(VB // 128):
            grp = i_ref[:, g * 128:(g + 1) * 128]
            for q in range(PACK):
                idxb = jnp.broadcast_to((idx + q)[None, :], (D, 128 // PACK))
                sel = jnp.take_along_axis(grp, idxb, axis=1,
                                          mode="promise_in_bounds")
                o_ref[pl.ds(g * (128 // PACK), 128 // PACK),
                      q * D:(q + 1) * D] = sel.T

    return pl.pallas_call(
        body,
        grid=(grid,),
        in_specs=[pl.BlockSpec((D, VB), lambda i: (0, i))],
        out_specs=pl.BlockSpec((VB * D // 128, 128), lambda i: (i, 0)),
        out_shape=jax.ShapeDtypeStruct((V * D // 128, 128), jnp.float32),
    )(tt)


def _detile_table(tt, tail_lin):
    D, V = tt.shape
    W = 128
    BO = W * D
    NBLK = V // W
    TAIL = V - NBLK * W
    TBO = TAIL * D
    mesh = plsc.VectorSubcoreMesh(core_axis_name="c", subcore_axis_name="s")
    nwork = NC * NS
    total_blocks = NBLK
    per_worker = (total_blocks + nwork - 1) // nwork
    NBUF = 2

    @functools.partial(
        pl.kernel,
        out_type=jax.ShapeDtypeStruct((V * D,), jnp.float32),
        mesh=mesh,
        compiler_params=pltpu.CompilerParams(use_tc_tiling_on_sc=True),
        scratch_types=dict(
            sin_v=[pltpu.VMEM((D, W), jnp.float32) for _ in range(NBUF)],
            sout_v=[pltpu.VMEM((BO,), jnp.float32) for _ in range(NBUF)],
            tail_vm=pltpu.VMEM((max(TBO, 8),), jnp.float32),
            isem=[pltpu.SemaphoreType.DMA for _ in range(NBUF)],
            osem=[pltpu.SemaphoreType.DMA for _ in range(NBUF)],
        ),
    )
    def tk(tt_hbm, tail_hbm, out_hbm, *, sin_v, sout_v, tail_vm, isem, osem):
        wid = lax.axis_index("s") * NC + lax.axis_index("c")
        lane_iota = lax.iota(jnp.int32, LANE)

        if TAIL:
            @pl.when(wid == nwork - 1)
            def _tail_copy():
                pltpu.sync_copy(tail_hbm, tail_vm.at[pl.ds(0, TBO)])
                pltpu.sync_copy(tail_vm.at[pl.ds(0, TBO)],
                                out_hbm.at[pl.ds(NBLK * BO, TBO)])

        def blk_id(t):
            return t * nwork + wid

        def issue(t, p):
            blk = blk_id(t)

            @pl.when(blk < NBLK)
            def _full():
                pltpu.async_copy(tt_hbm.at[:, pl.ds(blk * W, W)], sin_v[p],
                                 isem[p])

        def wait_in(t, p):
            blk = blk_id(t)

            @pl.when(blk < NBLK)
            def _full():
                pltpu.make_async_copy(tt_hbm.at[:, pl.ds(blk * W, W)],
                                      sin_v[p], isem[p]).wait()

        def compute(p):
            sin = sin_v[p]
            sout = sout_v[p]

            def cg_body(cg, _):
                c0 = cg * LANE
                for dg in range(D // LANE):
                    xs = [sin[dg * LANE + i, pl.ds(c0, LANE)]
                          for i in range(LANE)]
                    for s in (8, 4, 2, 1):
                        perm_idx = jnp.bitwise_xor(lane_iota, s)
                        new = []
                        for i in range(LANE):
                            pj = jnp.take_along_axis(
                                xs[i ^ s], perm_idx, axis=0,
                                mode="promise_in_bounds")
                            keep = (lane_iota & s) == (i & s)
                            new.append(jnp.where(keep, xs[i], pj))
                        xs = new
                    for c in range(LANE):
                        sout[pl.ds((c0 + c) * D + dg * LANE, LANE)] = xs[c]
                return 0

            lax.fori_loop(0, W // LANE, cg_body, 0)

        def write_out(t, p):
            blk = blk_id(t)

            @pl.when(blk < NBLK)
            def _full():
                pltpu.async_copy(sout_v[p],
                                 out_hbm.at[pl.ds(blk * BO, BO)], osem[p])

        def wait_out(t, p):
            blk = blk_id(t)

            @pl.when(blk < NBLK)
            def _full():
                pltpu.make_async_copy(
                    sout_v[p], out_hbm.at[pl.ds(blk * BO, BO)],
                    osem[p]).wait()

        for t0 in range(min(NBUF, per_worker)):
            @pl.when(blk_id(t0) < total_blocks)
            def _p(t0=t0):
                issue(t0, t0)

        def loop_body(t, _):
            p_sel = t % NBUF

            @pl.when(blk_id(t) < total_blocks)
            def _go():
                for p in range(NBUF):
                    @pl.when(p_sel == p)
                    def _b(p=p):
                        wait_in(t, p)

                        @pl.when(t >= NBUF)
                        def _wo():
                            wait_out(t - NBUF, p)

                        compute(p)
                        write_out(t, p)

                        @pl.when(blk_id(t + NBUF) < total_blocks)
                        def _nx():
                            issue(t + NBUF, p)
            return 0

        lax.fori_loop(0, per_worker, loop_body, 0)

        for dt in range(NBUF):
            t_last = per_worker - NBUF + dt
            if t_last >= 0:
                p = t_last % NBUF

                @pl.when(blk_id(t_last) < total_blocks)
                def _d(t_last=t_last, p=p):
                    wait_out(t_last, p)

    return tk(tt, tail_lin)


def _sent_vec_tfidf(words, table, TI):
    B, L = words.shape
    V, D = table.shape
    RB = B // NW
    CH = 32
    NCHUNK = RB // CH
    NBUF = 2

    mesh = plsc.VectorSubcoreMesh(core_axis_name="c", subcore_axis_name="s")

    @functools.partial(
        pl.kernel,
        out_type=jax.ShapeDtypeStruct((B, D), jnp.float32),
        mesh=mesh,
        compiler_params=pltpu.CompilerParams(use_tc_tiling_on_sc=False),
        scratch_types=dict(
            idx_v=[pltpu.VMEM((CH, L), jnp.int32) for _ in range(NBUF)],
            rows_v=[pltpu.VMEM((CH, L, D), jnp.float32) for _ in range(NBUF)],
            tiv_v=[pltpu.VMEM((CH, L), jnp.float32) for _ in range(NBUF)],
            outs_v=[pltpu.VMEM((CH, D), jnp.float32) for _ in range(NBUF)],
            rsem=[pltpu.SemaphoreType.DMA for _ in range(NBUF)],
            tsem=[pltpu.SemaphoreType.DMA for _ in range(NBUF)],
            osem=[pltpu.SemaphoreType.DMA for _ in range(NBUF)],
        ),
    )
    def k(words_hbm, table_hbm, ti_hbm, out_hbm, *, idx_v, rows_v, tiv_v,
          outs_v, rsem, tsem, osem):
        wid = lax.axis_index("s") * NC + lax.axis_index("c")
        row0 = wid * RB

        def issue(c, p):
            base = row0 + c * CH
            pltpu.sync_copy(words_hbm.at[pl.ds(base, CH)], idx_v[p])

            def row_issue(r, _):
                idx_r = idx_v[p].at[r]
                pltpu.async_copy(table_hbm.at[idx_r], rows_v[p].at[r],
                                 rsem[p])
                pltpu.async_copy(ti_hbm.at[idx_r], tiv_v[p].at[r], tsem[p])
                return 0

            lax.fori_loop(0, CH, row_issue, 0)

        def wait_gathers(p):
            def row_wait(r, _):
                idx_r = idx_v[p].at[r]
                pltpu.make_async_copy(table_hbm.at[idx_r], rows_v[p].at[r],
                                      rsem[p]).wait()
                pltpu.make_async_copy(ti_hbm.at[idx_r], tiv_v[p].at[r],
                                      tsem[p]).wait()
                return 0

            lax.fori_loop(0, CH, row_wait, 0)

        lane_iota = lax.iota(jnp.int32, LANE)

        def compute(p):
            tiv = tiv_v[p]
            rows = rows_v[p]
            outs = outs_v[p]

            def row_body(r, _):
                w0 = tiv[r, pl.ds(0, LANE)]
                w1 = tiv[r, pl.ds(16, LANE)]
                w2 = tiv[r, pl.ds(32, LANE)]
                w3 = tiv[r, pl.ds(L - LANE, LANE)]
                w3m = jnp.where(lane_iota >= (48 - (L - LANE)), w3, 0.0)
                wv = w0 + w1 + w2 + w3m
                for sh in (1, 2, 4, 8):
                    wv = wv + jnp.take_along_axis(
                        wv, jnp.bitwise_xor(lane_iota, sh), axis=0,
                        mode="promise_in_bounds")
                inv = 1.0 / (wv + 1e-8)

                chunks = (w0, w1, w2, w3)
                acc0 = jnp.zeros((LANE,), jnp.float32)
                acc1 = jnp.zeros((LANE,), jnp.float32)
                for l in range(L):
                    if l < 48:
                        cidx, lane = l // LANE, l % LANE
                    else:
                        cidx, lane = 3, l - (L - LANE)
                    wl = jnp.take_along_axis(
                        chunks[cidx], jnp.full((LANE,), lane, jnp.int32),
                        axis=0, mode="promise_in_bounds")
                    r0 = rows[r, l, pl.ds(0, LANE)]
                    r1 = rows[r, l, pl.ds(LANE, LANE)]
                    acc0 = acc0 + wl * r0
                    acc1 = acc1 + wl * r1
                outs[r, pl.ds(0, LANE)] = acc0 * inv
                outs[r, pl.ds(LANE, LANE)] = acc1 * inv
                return 0

            lax.fori_loop(0, CH, row_body, 0)

        for p in range(min(NBUF, NCHUNK)):
            issue(p, p)

        for c in range(NCHUNK):
            p = c % NBUF
            base = row0 + c * CH
            wait_gathers(p)
            if c >= NBUF:
                pltpu.make_async_copy(
                    outs_v[p], out_hbm.at[pl.ds(base - NBUF * CH, CH)],
                    osem[p]).wait()
            compute(p)
            pltpu.async_copy(outs_v[p], out_hbm.at[pl.ds(base, CH)], osem[p])
            nxt = c + NBUF
            if nxt < NCHUNK:
                issue(nxt, p)

        for c in range(max(0, NCHUNK - NBUF), NCHUNK):
            p = c % NBUF
            base = row0 + c * CH
            pltpu.make_async_copy(outs_v[p], out_hbm.at[pl.ds(base, CH)],
                                  osem[p]).wait()

    return k(words, table, TI)


def kernel(words, table, TI):
    V, D = table.shape
    tl = _detile_table_tc(table.T)
    tlin = tl.reshape(V, D)
    return _sent_vec_tfidf(words.astype(jnp.int32), tlin, TI)


# speedup vs baseline: 6.4041x; 6.4041x over previous
import functools

import jax
import jax.numpy as jnp
from jax import lax
from jax.experimental import pallas as pl
from jax.experimental.pallas import tpu as pltpu
from jax.experimental.pallas import tpu_sc as plsc

NC = 2
NS = 16
NW = NC * NS
LANE = 16


def _detile_table(tt, tail_lin):
    D, V = tt.shape
    W = 128
    BO = W * D
    NBLK = V // W
    TAIL = V - NBLK * W
    TBO = TAIL * D
    mesh = plsc.VectorSubcoreMesh(core_axis_name="c", subcore_axis_name="s")
    nwork = NC * NS
    total_blocks = NBLK
    per_worker = (total_blocks + nwork - 1) // nwork
    NBUF = 2

    @functools.partial(
        pl.kernel,
        out_type=jax.ShapeDtypeStruct((V * D,), jnp.float32),
        mesh=mesh,
        compiler_params=pltpu.CompilerParams(use_tc_tiling_on_sc=True),
        scratch_types=dict(
            sin_v=[pltpu.VMEM((D, W), jnp.float32) for _ in range(NBUF)],
            sout_v=[pltpu.VMEM((BO,), jnp.float32) for _ in range(NBUF)],
            tail_vm=pltpu.VMEM((max(TBO, 8),), jnp.float32),
            isem=[pltpu.SemaphoreType.DMA for _ in range(NBUF)],
            osem=[pltpu.SemaphoreType.DMA for _ in range(NBUF)],
        ),
    )
    def tk(tt_hbm, tail_hbm, out_hbm, *, sin_v, sout_v, tail_vm, isem, osem):
        wid = lax.axis_index("s") * NC + lax.axis_index("c")
        lane_iota = lax.iota(jnp.int32, LANE)

        if TAIL:
            @pl.when(wid == nwork - 1)
            def _tail_copy():
                pltpu.sync_copy(tail_hbm, tail_vm.at[pl.ds(0, TBO)])
                pltpu.sync_copy(tail_vm.at[pl.ds(0, TBO)],
                                out_hbm.at[pl.ds(NBLK * BO, TBO)])

        def blk_id(t):
            return t * nwork + wid

        def issue(t, p):
            blk = blk_id(t)

            @pl.when(blk < NBLK)
            def _full():
                pltpu.async_copy(tt_hbm.at[:, pl.ds(blk * W, W)], sin_v[p],
                                 isem[p])

        def wait_in(t, p):
            blk = blk_id(t)

            @pl.when(blk < NBLK)
            def _full():
                pltpu.make_async_copy(tt_hbm.at[:, pl.ds(blk * W, W)],
                                      sin_v[p], isem[p]).wait()

        def compute(p):
            sin = sin_v[p]
            sout = sout_v[p]

            for cg in range(W // LANE):
                c0 = cg * LANE
                for dg in range(D // LANE):
                    xs = [sin[dg * LANE + i, pl.ds(c0, LANE)]
                          for i in range(LANE)]
                    for s in (8, 4, 2, 1):
                        perm_idx = jnp.bitwise_xor(lane_iota, s)
                        new = []
                        for i in range(LANE):
                            pj = jnp.take_along_axis(
                                xs[i ^ s], perm_idx, axis=0,
                                mode="promise_in_bounds")
                            keep = (lane_iota & s) == (i & s)
                            new.append(jnp.where(keep, xs[i], pj))
                        xs = new
                    for c in range(LANE):
                        sout[pl.ds((c0 + c) * D + dg * LANE, LANE)] = xs[c]

        def write_out(t, p):
            blk = blk_id(t)

            @pl.when(blk < NBLK)
            def _full():
                pltpu.async_copy(sout_v[p],
                                 out_hbm.at[pl.ds(blk * BO, BO)], osem[p])

        def wait_out(t, p):
            blk = blk_id(t)

            @pl.when(blk < NBLK)
            def _full():
                pltpu.make_async_copy(
                    sout_v[p], out_hbm.at[pl.ds(blk * BO, BO)],
                    osem[p]).wait()

        for t0 in range(min(NBUF, per_worker)):
            @pl.when(blk_id(t0) < total_blocks)
            def _p(t0=t0):
                issue(t0, t0)

        def loop_body(t, _):
            p_sel = t % NBUF

            @pl.when(blk_id(t) < total_blocks)
            def _go():
                for p in range(NBUF):
                    @pl.when(p_sel == p)
                    def _b(p=p):
                        wait_in(t, p)

                        @pl.when(t >= NBUF)
                        def _wo():
                            wait_out(t - NBUF, p)

                        compute(p)
                        write_out(t, p)

                        @pl.when(blk_id(t + NBUF) < total_blocks)
                        def _nx():
                            issue(t + NBUF, p)
            return 0

        lax.fori_loop(0, per_worker, loop_body, 0)

        for dt in range(NBUF):
            t_last = per_worker - NBUF + dt
            if t_last >= 0:
                p = t_last % NBUF

                @pl.when(blk_id(t_last) < total_blocks)
                def _d(t_last=t_last, p=p):
                    wait_out(t_last, p)

    return tk(tt, tail_lin)


def _sent_vec_tfidf(words, table, TI):
    B, L = words.shape
    V, D = table.shape
    RB = B // NW
    CH = 32
    NCHUNK = RB // CH
    NBUF = 2

    mesh = plsc.VectorSubcoreMesh(core_axis_name="c", subcore_axis_name="s")

    @functools.partial(
        pl.kernel,
        out_type=jax.ShapeDtypeStruct((B, D), jnp.float32),
        mesh=mesh,
        compiler_params=pltpu.CompilerParams(use_tc_tiling_on_sc=False),
        scratch_types=dict(
            idx_v=[pltpu.VMEM((CH, L), jnp.int32) for _ in range(NBUF)],
            rows_v=[pltpu.VMEM((CH, L, D), jnp.float32) for _ in range(NBUF)],
            tiv_v=[pltpu.VMEM((CH, L), jnp.float32) for _ in range(NBUF)],
            outs_v=[pltpu.VMEM((CH, D), jnp.float32) for _ in range(NBUF)],
            rsem=[pltpu.SemaphoreType.DMA for _ in range(NBUF)],
            tsem=[pltpu.SemaphoreType.DMA for _ in range(NBUF)],
            osem=[pltpu.SemaphoreType.DMA for _ in range(NBUF)],
        ),
    )
    def k(words_hbm, table_hbm, ti_hbm, out_hbm, *, idx_v, rows_v, tiv_v,
          outs_v, rsem, tsem, osem):
        wid = lax.axis_index("s") * NC + lax.axis_index("c")
        row0 = wid * RB

        def issue(c, p):
            base = row0 + c * CH
            pltpu.sync_copy(words_hbm.at[pl.ds(base, CH)], idx_v[p])

            def row_issue(r, _):
                idx_r = idx_v[p].at[r]
                pltpu.async_copy(table_hbm.at[idx_r], rows_v[p].at[r],
                                 rsem[p])
                pltpu.async_copy(ti_hbm.at[idx_r], tiv_v[p].at[r], tsem[p])
                return 0

            lax.fori_loop(0, CH, row_issue, 0)

        def wait_gathers(p):
            def row_wait(r, _):
                idx_r = idx_v[p].at[r]
                pltpu.make_async_copy(table_hbm.at[idx_r], rows_v[p].at[r],
                                      rsem[p]).wait()
                pltpu.make_async_copy(ti_hbm.at[idx_r], tiv_v[p].at[r],
                                      tsem[p]).wait()
                return 0

            lax.fori_loop(0, CH, row_wait, 0)

        lane_iota = lax.iota(jnp.int32, LANE)

        def compute(p):
            tiv = tiv_v[p]
            rows = rows_v[p]
            outs = outs_v[p]

            def row_body(r, _):
                w0 = tiv[r, pl.ds(0, LANE)]
                w1 = tiv[r, pl.ds(16, LANE)]
                w2 = tiv[r, pl.ds(32, LANE)]
                w3 = tiv[r, pl.ds(L - LANE, LANE)]
                w3m = jnp.where(lane_iota >= (48 - (L - LANE)), w3, 0.0)
                wv = w0 + w1 + w2 + w3m
                for sh in (1, 2, 4, 8):
                    wv = wv + jnp.take_along_axis(
                        wv, jnp.bitwise_xor(lane_iota, sh), axis=0,
                        mode="promise_in_bounds")
                inv = 1.0 / (wv + 1e-8)

                chunks = (w0, w1, w2, w3)
                acc0 = jnp.zeros((LANE,), jnp.float32)
                acc1 = jnp.zeros((LANE,), jnp.float32)
                for l in range(L):
                    if l < 48:
                        cidx, lane = l // LANE, l % LANE
                    else:
                        cidx, lane = 3, l - (L - LANE)
                    wl = jnp.take_along_axis(
                        chunks[cidx], jnp.full((LANE,), lane, jnp.int32),
                        axis=0, mode="promise_in_bounds")
                    r0 = rows[r, l, pl.ds(0, LANE)]
                    r1 = rows[r, l, pl.ds(LANE, LANE)]
                    acc0 = acc0 + wl * r0
                    acc1 = acc1 + wl * r1
                outs[r, pl.ds(0, LANE)] = acc0 * inv
                outs[r, pl.ds(LANE, LANE)] = acc1 * inv
                return 0

            lax.fori_loop(0, CH, row_body, 0)

        for p in range(min(NBUF, NCHUNK)):
            issue(p, p)

        for c in range(NCHUNK):
            p = c % NBUF
            base = row0 + c * CH
            wait_gathers(p)
            if c >= NBUF:
                pltpu.make_async_copy(
                    outs_v[p], out_hbm.at[pl.ds(base - NBUF * CH, CH)],
                    osem[p]).wait()
            compute(p)
            pltpu.async_copy(outs_v[p], out_hbm.at[pl.ds(base, CH)], osem[p])
            nxt = c + NBUF
            if nxt < NCHUNK:
                issue(nxt, p)

        for c in range(max(0, NCHUNK - NBUF), NCHUNK):
            p = c % NBUF
            base = row0 + c * CH
            pltpu.make_async_copy(outs_v[p], out_hbm.at[pl.ds(base, CH)],
                                  osem[p]).wait()

    return k(words, table, TI)


def kernel(words, table, TI):
    V, D = table.shape
    tail = V % 128
    tail_lin = table[V - tail:].reshape(-1) if tail else jnp.zeros(
        (8,), jnp.float32)
    tl = _detile_table(table.T, tail_lin)
    tlin = tl.reshape(V, D)
    return _sent_vec_tfidf(words.astype(jnp.int32), tlin, TI)


# speedup vs baseline: 9.8967x; 1.5454x over previous
import functools

import jax
import jax.numpy as jnp
from jax import lax
from jax.experimental import pallas as pl
from jax.experimental.pallas import tpu as pltpu
from jax.experimental.pallas import tpu_sc as plsc

NC = 2
NS = 16
NW = NC * NS
LANE = 16


def _detile_table(tt, tail_lin):
    D, V = tt.shape
    W = 128
    BO = W * D
    NBLK = V // W
    TAIL = V - NBLK * W
    TBO = TAIL * D
    mesh = plsc.VectorSubcoreMesh(core_axis_name="c", subcore_axis_name="s")
    nwork = NC * NS
    total_blocks = NBLK
    per_worker = (total_blocks + nwork - 1) // nwork
    NBUF = 2

    @functools.partial(
        pl.kernel,
        out_type=jax.ShapeDtypeStruct((V * D,), jnp.float32),
        mesh=mesh,
        compiler_params=pltpu.CompilerParams(use_tc_tiling_on_sc=True),
        scratch_types=dict(
            sin_v=[pltpu.VMEM((D, W), jnp.float32) for _ in range(NBUF)],
            sout_v=[pltpu.VMEM((BO,), jnp.float32) for _ in range(NBUF)],
            tail_vm=pltpu.VMEM((max(TBO, 8),), jnp.float32),
            isem=[pltpu.SemaphoreType.DMA for _ in range(NBUF)],
            osem=[pltpu.SemaphoreType.DMA for _ in range(NBUF)],
        ),
    )
    def tk(tt_hbm, tail_hbm, out_hbm, *, sin_v, sout_v, tail_vm, isem, osem):
        wid = lax.axis_index("s") * NC + lax.axis_index("c")
        lane_iota = lax.iota(jnp.int32, LANE)

        if TAIL:
            @pl.when(wid == nwork - 1)
            def _tail_copy():
                pltpu.sync_copy(tail_hbm, tail_vm.at[pl.ds(0, TBO)])
                pltpu.sync_copy(tail_vm.at[pl.ds(0, TBO)],
                                out_hbm.at[pl.ds(NBLK * BO, TBO)])

        def blk_id(t):
            return t * nwork + wid

        def issue(t, p):
            blk = blk_id(t)

            @pl.when(blk < NBLK)
            def _full():
                pltpu.async_copy(tt_hbm.at[:, pl.ds(blk * W, W)], sin_v[p],
                                 isem[p])

        def wait_in(t, p):
            blk = blk_id(t)

            @pl.when(blk < NBLK)
            def _full():
                pltpu.make_async_copy(tt_hbm.at[:, pl.ds(blk * W, W)],
                                      sin_v[p], isem[p]).wait()

        def compute(p):
            sin = sin_v[p]
            sout = sout_v[p]

            def cg_body(cg, _):
                c0 = cg * LANE
                for dg in range(D // LANE):
                    xs = [sin[dg * LANE + i, pl.ds(c0, LANE)]
                          for i in range(LANE)]
                    for s in (8, 4, 2, 1):
                        perm_idx = jnp.bitwise_xor(lane_iota, s)
                        new = []
                        for i in range(LANE):
                            pj = jnp.take_along_axis(
                                xs[i ^ s], perm_idx, axis=0,
                                mode="promise_in_bounds")
                            keep = (lane_iota & s) == (i & s)
                            new.append(jnp.where(keep, xs[i], pj))
                        xs = new
                    for c in range(LANE):
                        sout[pl.ds((c0 + c) * D + dg * LANE, LANE)] = xs[c]
                return 0

            lax.fori_loop(0, W // LANE, cg_body, 0)

        def write_out(t, p):
            blk = blk_id(t)

            @pl.when(blk < NBLK)
            def _full():
                pltpu.async_copy(sout_v[p],
                                 out_hbm.at[pl.ds(blk * BO, BO)], osem[p])

        def wait_out(t, p):
            blk = blk_id(t)

            @pl.when(blk < NBLK)
            def _full():
                pltpu.make_async_copy(
                    sout_v[p], out_hbm.at[pl.ds(blk * BO, BO)],
                    osem[p]).wait()

        for t0 in range(min(NBUF, per_worker)):
            @pl.when(blk_id(t0) < total_blocks)
            def _p(t0=t0):
                issue(t0, t0)

        def loop_body(t, _):
            p_sel = t % NBUF

            @pl.when(blk_id(t) < total_blocks)
            def _go():
                for p in range(NBUF):
                    @pl.when(p_sel == p)
                    def _b(p=p):
                        wait_in(t, p)

                        @pl.when(t >= NBUF)
                        def _wo():
                            wait_out(t - NBUF, p)

                        compute(p)
                        write_out(t, p)

                        @pl.when(blk_id(t + NBUF) < total_blocks)
                        def _nx():
                            issue(t + NBUF, p)
            return 0

        lax.fori_loop(0, per_worker, loop_body, 0)

        for dt in range(NBUF):
            t_last = per_worker - NBUF + dt
            if t_last >= 0:
                p = t_last % NBUF

                @pl.when(blk_id(t_last) < total_blocks)
                def _d(t_last=t_last, p=p):
                    wait_out(t_last, p)

    return tk(tt, tail_lin)


def _sent_vec_tfidf(words, table, TI):
    B, L = words.shape
    V, D = table.shape
    RB = B // NW
    CH = 32
    NCHUNK = RB // CH
    NBUF = 2

    mesh = plsc.VectorSubcoreMesh(core_axis_name="c", subcore_axis_name="s")

    @functools.partial(
        pl.kernel,
        out_type=jax.ShapeDtypeStruct((B, D), jnp.float32),
        mesh=mesh,
        compiler_params=pltpu.CompilerParams(use_tc_tiling_on_sc=False),
        scratch_types=dict(
            idx_v=[pltpu.VMEM((CH, L), jnp.int32) for _ in range(NBUF)],
            rows_v=[pltpu.VMEM((CH, L, D), jnp.float32) for _ in range(NBUF)],
            tiv_v=[pltpu.VMEM((CH, L), jnp.float32) for _ in range(NBUF)],
            outs_v=[pltpu.VMEM((CH, D), jnp.float32) for _ in range(NBUF)],
            rsem=[pltpu.SemaphoreType.DMA for _ in range(NBUF)],
            tsem=[pltpu.SemaphoreType.DMA for _ in range(NBUF)],
            osem=[pltpu.SemaphoreType.DMA for _ in range(NBUF)],
        ),
    )
    def k(words_hbm, table_hbm, ti_hbm, out_hbm, *, idx_v, rows_v, tiv_v,
          outs_v, rsem, tsem, osem):
        wid = lax.axis_index("s") * NC + lax.axis_index("c")
        row0 = wid * RB

        def issue(c, p):
            base = row0 + c * CH
            pltpu.sync_copy(words_hbm.at[pl.ds(base, CH)], idx_v[p])

            def row_issue(r, _):
                idx_r = idx_v[p].at[r]
                pltpu.async_copy(table_hbm.at[idx_r], rows_v[p].at[r],
                                 rsem[p])
                pltpu.async_copy(ti_hbm.at[idx_r], tiv_v[p].at[r], tsem[p])
                return 0

            lax.fori_loop(0, CH, row_issue, 0)

        def wait_gathers(p):
            def row_wait(r, _):
                idx_r = idx_v[p].at[r]
                pltpu.make_async_copy(table_hbm.at[idx_r], rows_v[p].at[r],
                                      rsem[p]).wait()
                pltpu.make_async_copy(ti_hbm.at[idx_r], tiv_v[p].at[r],
                                      tsem[p]).wait()
                return 0

            lax.fori_loop(0, CH, row_wait, 0)

        lane_iota = lax.iota(jnp.int32, LANE)

        def compute(p):
            tiv = tiv_v[p]
            rows = rows_v[p]
            outs = outs_v[p]

            def row_body(r, _):
                w0 = tiv[r, pl.ds(0, LANE)]
                w1 = tiv[r, pl.ds(16, LANE)]
                w2 = tiv[r, pl.ds(32, LANE)]
                w3 = tiv[r, pl.ds(L - LANE, LANE)]
                w3m = jnp.where(lane_iota >= (48 - (L - LANE)), w3, 0.0)
                wv = w0 + w1 + w2 + w3m
                for sh in (1, 2, 4, 8):
                    wv = wv + jnp.take_along_axis(
                        wv, jnp.bitwise_xor(lane_iota, sh), axis=0,
                        mode="promise_in_bounds")
                inv = 1.0 / (wv + 1e-8)

                chunks = (w0, w1, w2, w3)
                acc0 = jnp.zeros((LANE,), jnp.float32)
                acc1 = jnp.zeros((LANE,), jnp.float32)
                for l in range(L):
                    if l < 48:
                        cidx, lane = l // LANE, l % LANE
                    else:
                        cidx, lane = 3, l - (L - LANE)
                    wl = jnp.take_along_axis(
                        chunks[cidx], jnp.full((LANE,), lane, jnp.int32),
                        axis=0, mode="promise_in_bounds")
                    r0 = rows[r, l, pl.ds(0, LANE)]
                    r1 = rows[r, l, pl.ds(LANE, LANE)]
                    acc0 = acc0 + wl * r0
                    acc1 = acc1 + wl * r1
                outs[r, pl.ds(0, LANE)] = acc0 * inv
                outs[r, pl.ds(LANE, LANE)] = acc1 * inv
                return 0

            lax.fori_loop(0, CH, row_body, 0)

        for p in range(min(NBUF, NCHUNK)):
            issue(p, p)

        for c in range(NCHUNK):
            p = c % NBUF
            base = row0 + c * CH
            wait_gathers(p)
            if c >= NBUF:
                pltpu.make_async_copy(
                    outs_v[p], out_hbm.at[pl.ds(base - NBUF * CH, CH)],
                    osem[p]).wait()
            compute(p)
            pltpu.async_copy(outs_v[p], out_hbm.at[pl.ds(base, CH)], osem[p])
            nxt = c + NBUF
            if nxt < NCHUNK:
                issue(nxt, p)

        for c in range(max(0, NCHUNK - NBUF), NCHUNK):
            p = c % NBUF
            base = row0 + c * CH
            pltpu.make_async_copy(outs_v[p], out_hbm.at[pl.ds(base, CH)],
                                  osem[p]).wait()

    return k(words, table, TI)


def kernel(words, table, TI):
    V, D = table.shape
    tail = V % 128
    tail_lin = table[V - tail:].reshape(-1) if tail else jnp.zeros(
        (8,), jnp.float32)
    tl = _detile_table(table.T, tail_lin)
    tlin = tl.reshape(V, D)
    return _sent_vec_tfidf(words.astype(jnp.int32), tlin, TI)
